# baseline (speedup 1.0000x reference)
import functools

import jax
import jax.numpy as jnp
from jax.experimental import pallas as pl
from jax.experimental.pallas import tpu as pltpu

N = 100000
B = 64
C_IN = 256
C_MID = 256
C_OUT = 512

R = 10000
NB = N // R
SUB = 1000
NSUB = R // SUB
NT = N // SUB
RC = 5000
NC = N // RC

_NEG_INF = float("-inf")


def _center_body(ids_ref, pos_ref, w1b_ref, b1_ref, cadj_ref, acc_pos, acc_cnt):
    i = pl.program_id(0)

    @pl.when(i == 0)
    def _():
        acc_pos[...] = jnp.zeros_like(acc_pos)
        acc_cnt[...] = jnp.zeros_like(acc_cnt)

    ids_row = ids_ref[0]
    onehot = (jax.lax.broadcasted_iota(jnp.int32, (B, RC), 0) == ids_row
              ).astype(jnp.float32)
    acc_pos[...] += jax.lax.dot_general(
        onehot, pos_ref[...], (((1,), (0,)), ((), ())),
        preferred_element_type=jnp.float32)
    acc_cnt[...] += jax.lax.dot_general(
        onehot, jnp.ones((RC, 1), jnp.float32), (((1,), (0,)), ((), ())),
        preferred_element_type=jnp.float32)

    @pl.when(i == NC - 1)
    def _():
        center = acc_pos[...] / jnp.maximum(acc_cnt[...], 1.0)
        cadj_ref[...] = b1_ref[...] - jax.lax.dot_general(
            center, w1b_ref[...], (((1,), (0,)), ((), ())),
            preferred_element_type=jnp.float32)


def _main_body(bounds_ref, ids_ref, feat_ref, pos_ref, w1a_ref, w1b_ref,
               aggr_ref):
    i = pl.program_id(0)

    @pl.when(i == 0)
    def _():
        aggr_ref[...] = jnp.full((B, C_MID), _NEG_INF, jnp.float32)

    for t in range(NSUB):
        ids_t = ids_ref[0, t * SUB:(t + 1) * SUB]
        a_t = jax.lax.dot_general(
            feat_ref[t * SUB:(t + 1) * SUB, :].astype(jnp.bfloat16),
            w1a_ref[...], (((1,), (0,)), ((), ())),
            preferred_element_type=jnp.float32)
        a_t += jax.lax.dot_general(
            pos_ref[t * SUB:(t + 1) * SUB, :].astype(jnp.bfloat16),
            w1b_ref[...], (((1,), (0,)), ((), ())),
            preferred_element_type=jnp.float32)

        s_lo = bounds_ref[0, i * NSUB + t]
        s_hi = bounds_ref[1, i * NSUB + t]

        def seg_step(s, carry, ids_t=ids_t, a_t=a_t):
            red = jnp.max(jnp.where(ids_t == s, a_t, _NEG_INF), axis=0,
                          keepdims=True)
            aggr_ref[pl.ds(s, 1), :] = jnp.maximum(aggr_ref[pl.ds(s, 1), :],
                                                   red)
            return carry

        jax.lax.fori_loop(s_lo, s_hi + 1, seg_step, 0)


def _epi_body(aggr_ref, cadj_ref, w2_ref, b2_ref, eps_ref, z_ref, mu_ref, sig_ref):
    agg = jnp.maximum(aggr_ref[...] + cadj_ref[...], 0.0)
    out = jax.lax.dot_general(
        agg, w2_ref[...], (((1,), (0,)), ((), ())),
        preferred_element_type=jnp.float32) + b2_ref[...]
    mu = out[:, :C_MID]
    sr = out[:, C_MID:]
    sigma = jnp.maximum(sr, 0.0) + jnp.log1p(jnp.exp(-jnp.abs(sr)))
    mu_ref[...] = mu
    sig_ref[...] = sigma
    z_ref[...] = mu + sigma * eps_ref[...]


@functools.partial(jax.jit, static_argnames=("interpret",))
def _run(pos, feature, ids_col, ids_row, bounds, W1a, W1b, b1r, W2, b2r, eps,
         interpret=False):
    cadj = pl.pallas_call(
        _center_body,
        grid=(NC,),
        in_specs=[
            pl.BlockSpec((1, 1, RC), lambda i: (i, 0, 0)),
            pl.BlockSpec((RC, 3), lambda i: (i, 0)),
            pl.BlockSpec((3, C_MID), lambda i: (0, 0)),
            pl.BlockSpec((1, C_MID), lambda i: (0, 0)),
        ],
        out_specs=pl.BlockSpec((B, C_MID), lambda i: (0, 0)),
        out_shape=jax.ShapeDtypeStruct((B, C_MID), jnp.float32),
        scratch_shapes=[
            pltpu.VMEM((B, 3), jnp.float32),
            pltpu.VMEM((B, 1), jnp.float32),
        ],
        interpret=interpret,
    )(ids_row, pos, W1b, b1r)

    agg_raw = pl.pallas_call(
        _main_body,
        grid=(NB,),
        in_specs=[
            pl.BlockSpec(memory_space=pltpu.SMEM),
            pl.BlockSpec((1, R, 1), lambda i: (i, 0, 0)),
            pl.BlockSpec((R, C_IN), lambda i: (i, 0)),
            pl.BlockSpec((R, 3), lambda i: (i, 0)),
            pl.BlockSpec((C_IN, C_MID), lambda i: (0, 0)),
            pl.BlockSpec((3, C_MID), lambda i: (0, 0)),
        ],
        out_specs=pl.BlockSpec((B, C_MID), lambda i: (0, 0)),
        out_shape=jax.ShapeDtypeStruct((B, C_MID), jnp.float32),
        interpret=interpret,
    )(bounds, ids_col, feature, pos,
      W1a.astype(jnp.bfloat16), W1b.astype(jnp.bfloat16))

    z, mu, sigma = pl.pallas_call(
        _epi_body,
        in_specs=[
            pl.BlockSpec((B, C_MID), lambda: (0, 0)),
            pl.BlockSpec((B, C_MID), lambda: (0, 0)),
            pl.BlockSpec((C_MID, C_OUT), lambda: (0, 0)),
            pl.BlockSpec((1, C_OUT), lambda: (0, 0)),
            pl.BlockSpec((B, C_MID), lambda: (0, 0)),
        ],
        out_specs=[
            pl.BlockSpec((B, C_MID), lambda: (0, 0)),
            pl.BlockSpec((B, C_MID), lambda: (0, 0)),
            pl.BlockSpec((B, C_MID), lambda: (0, 0)),
        ],
        out_shape=[
            jax.ShapeDtypeStruct((B, C_MID), jnp.float32),
            jax.ShapeDtypeStruct((B, C_MID), jnp.float32),
            jax.ShapeDtypeStruct((B, C_MID), jnp.float32),
        ],
        interpret=interpret,
    )(agg_raw, cadj, W2, b2r, eps)
    return z, mu, sigma


def kernel(pos, feature, batch, W1, b1, W2, b2, *, interpret=False):
    ids = batch.astype(jnp.int32)
    ids_col = ids.reshape(NB, R, 1)
    ids_row = ids.reshape(NC, 1, RC)
    ids_sub = ids.reshape(NT, SUB)
    bounds = jnp.stack([ids_sub[:, 0], ids_sub[:, SUB - 1]])
    W1a = W1[:C_IN]
    W1b = W1[C_IN:]
    b1r = b1.reshape(1, C_MID)
    b2r = b2.reshape(1, C_OUT)
    eps = jax.random.normal(jax.random.key(1), (B, C_MID), dtype=jnp.float32)
    z, mu, sigma = _run(pos, feature, ids_col, ids_row, bounds, W1a, W1b, b1r,
                        W2, b2r, eps, interpret=interpret)
    pos_center_batch = jnp.arange(B, dtype=jnp.int64)
    return (z, mu, sigma, pos_center_batch)


# speedup vs baseline: 1.0341x; 1.0341x over previous
import functools

import jax
import jax.numpy as jnp
from jax.experimental import pallas as pl
from jax.experimental.pallas import tpu as pltpu

N = 100000
B = 64
C_IN = 256
C_MID = 256
C_OUT = 512

R = 2000
NB = N // R
SUB = 1000
NSUB = R // SUB
NT = N // SUB
RC = 5000
NC = N // RC

_NEG_INF = float("-inf")


def _center_body(ids_ref, pos_ref, w1b_ref, b1_ref, cadj_ref, acc_pos, acc_cnt):
    i = pl.program_id(0)

    @pl.when(i == 0)
    def _():
        acc_pos[...] = jnp.zeros_like(acc_pos)
        acc_cnt[...] = jnp.zeros_like(acc_cnt)

    ids_row = ids_ref[0]
    onehot = (jax.lax.broadcasted_iota(jnp.int32, (B, RC), 0) == ids_row
              ).astype(jnp.float32)
    acc_pos[...] += jax.lax.dot_general(
        onehot, pos_ref[...], (((1,), (0,)), ((), ())),
        preferred_element_type=jnp.float32)
    acc_cnt[...] += jax.lax.dot_general(
        onehot, jnp.ones((RC, 1), jnp.float32), (((1,), (0,)), ((), ())),
        preferred_element_type=jnp.float32)

    @pl.when(i == NC - 1)
    def _():
        center = acc_pos[...] / jnp.maximum(acc_cnt[...], 1.0)
        cadj_ref[...] = b1_ref[...] - jax.lax.dot_general(
            center, w1b_ref[...], (((1,), (0,)), ((), ())),
            preferred_element_type=jnp.float32)


def _main_body(bounds_ref, ids_ref, feat_ref, pos_ref, w1a_ref, w1b_ref,
               aggr_ref):
    i = pl.program_id(0)

    @pl.when(i == 0)
    def _():
        aggr_ref[...] = jnp.full((B, C_MID), _NEG_INF, jnp.float32)

    for t in range(NSUB):
        ids_t = ids_ref[0, t * SUB:(t + 1) * SUB]
        a_t = jax.lax.dot_general(
            feat_ref[t * SUB:(t + 1) * SUB, :].astype(jnp.bfloat16),
            w1a_ref[...], (((1,), (0,)), ((), ())),
            preferred_element_type=jnp.float32)
        a_t += jax.lax.dot_general(
            pos_ref[t * SUB:(t + 1) * SUB, :].astype(jnp.bfloat16),
            w1b_ref[...], (((1,), (0,)), ((), ())),
            preferred_element_type=jnp.float32)

        s_lo = bounds_ref[0, i * NSUB + t]
        s_hi = bounds_ref[1, i * NSUB + t]

        def seg_step(s, carry, ids_t=ids_t, a_t=a_t):
            red = jnp.max(jnp.where(ids_t == s, a_t, _NEG_INF), axis=0,
                          keepdims=True)
            aggr_ref[pl.ds(s, 1), :] = jnp.maximum(aggr_ref[pl.ds(s, 1), :],
                                                   red)
            return carry

        jax.lax.fori_loop(s_lo, s_hi + 1, seg_step, 0)


def _epi_body(aggr_ref, cadj_ref, w2_ref, b2_ref, eps_ref, z_ref, mu_ref, sig_ref):
    agg = jnp.maximum(aggr_ref[...] + cadj_ref[...], 0.0)
    out = jax.lax.dot_general(
        agg, w2_ref[...], (((1,), (0,)), ((), ())),
        preferred_element_type=jnp.float32) + b2_ref[...]
    mu = out[:, :C_MID]
    sr = out[:, C_MID:]
    sigma = jnp.maximum(sr, 0.0) + jnp.log1p(jnp.exp(-jnp.abs(sr)))
    mu_ref[...] = mu
    sig_ref[...] = sigma
    z_ref[...] = mu + sigma * eps_ref[...]


@functools.partial(jax.jit, static_argnames=("interpret",))
def _run(pos, feature, ids_col, ids_row, bounds, W1a, W1b, b1r, W2, b2r, eps,
         interpret=False):
    cadj = pl.pallas_call(
        _center_body,
        grid=(NC,),
        in_specs=[
            pl.BlockSpec((1, 1, RC), lambda i: (i, 0, 0)),
            pl.BlockSpec((RC, 3), lambda i: (i, 0)),
            pl.BlockSpec((3, C_MID), lambda i: (0, 0)),
            pl.BlockSpec((1, C_MID), lambda i: (0, 0)),
        ],
        out_specs=pl.BlockSpec((B, C_MID), lambda i: (0, 0)),
        out_shape=jax.ShapeDtypeStruct((B, C_MID), jnp.float32),
        scratch_shapes=[
            pltpu.VMEM((B, 3), jnp.float32),
            pltpu.VMEM((B, 1), jnp.float32),
        ],
        interpret=interpret,
    )(ids_row, pos, W1b, b1r)

    agg_raw = pl.pallas_call(
        _main_body,
        grid=(NB,),
        in_specs=[
            pl.BlockSpec(memory_space=pltpu.SMEM),
            pl.BlockSpec((1, R, 1), lambda i: (i, 0, 0)),
            pl.BlockSpec((R, C_IN), lambda i: (i, 0)),
            pl.BlockSpec((R, 3), lambda i: (i, 0)),
            pl.BlockSpec((C_IN, C_MID), lambda i: (0, 0)),
            pl.BlockSpec((3, C_MID), lambda i: (0, 0)),
        ],
        out_specs=pl.BlockSpec((B, C_MID), lambda i: (0, 0)),
        out_shape=jax.ShapeDtypeStruct((B, C_MID), jnp.float32),
        interpret=interpret,
    )(bounds, ids_col, feature, pos,
      W1a.astype(jnp.bfloat16), W1b.astype(jnp.bfloat16))

    z, mu, sigma = pl.pallas_call(
        _epi_body,
        in_specs=[
            pl.BlockSpec((B, C_MID), lambda: (0, 0)),
            pl.BlockSpec((B, C_MID), lambda: (0, 0)),
            pl.BlockSpec((C_MID, C_OUT), lambda: (0, 0)),
            pl.BlockSpec((1, C_OUT), lambda: (0, 0)),
            pl.BlockSpec((B, C_MID), lambda: (0, 0)),
        ],
        out_specs=[
            pl.BlockSpec((B, C_MID), lambda: (0, 0)),
            pl.BlockSpec((B, C_MID), lambda: (0, 0)),
            pl.BlockSpec((B, C_MID), lambda: (0, 0)),
        ],
        out_shape=[
            jax.ShapeDtypeStruct((B, C_MID), jnp.float32),
            jax.ShapeDtypeStruct((B, C_MID), jnp.float32),
            jax.ShapeDtypeStruct((B, C_MID), jnp.float32),
        ],
        interpret=interpret,
    )(agg_raw, cadj, W2, b2r, eps)
    return z, mu, sigma


def kernel(pos, feature, batch, W1, b1, W2, b2, *, interpret=False):
    ids = batch.astype(jnp.int32)
    ids_col = ids.reshape(NB, R, 1)
    ids_row = ids.reshape(NC, 1, RC)
    ids_sub = ids.reshape(NT, SUB)
    bounds = jnp.stack([ids_sub[:, 0], ids_sub[:, SUB - 1]])
    W1a = W1[:C_IN]
    W1b = W1[C_IN:]
    b1r = b1.reshape(1, C_MID)
    b2r = b2.reshape(1, C_OUT)
    eps = jax.random.normal(jax.random.key(1), (B, C_MID), dtype=jnp.float32)
    z, mu, sigma = _run(pos, feature, ids_col, ids_row, bounds, W1a, W1b, b1r,
                        W2, b2r, eps, interpret=interpret)
    pos_center_batch = jnp.arange(B, dtype=jnp.int64)
    return (z, mu, sigma, pos_center_batch)


# speedup vs baseline: 1.0922x; 1.0562x over previous
import functools

import jax
import jax.numpy as jnp
from jax.experimental import pallas as pl
from jax.experimental.pallas import tpu as pltpu

N = 100000
B = 64
C_IN = 256
C_MID = 256
C_OUT = 512

R = 2000
NB = N // R
SUB = 2000
NSUB = R // SUB
NT = N // SUB
RC = 5000
NC = N // RC

_NEG_INF = float("-inf")


def _center_body(ids_ref, pos_ref, w1b_ref, b1_ref, cadj_ref, acc_pos, acc_cnt):
    i = pl.program_id(0)

    @pl.when(i == 0)
    def _():
        acc_pos[...] = jnp.zeros_like(acc_pos)
        acc_cnt[...] = jnp.zeros_like(acc_cnt)

    ids_row = ids_ref[0]
    onehot = (jax.lax.broadcasted_iota(jnp.int32, (B, RC), 0) == ids_row
              ).astype(jnp.float32)
    acc_pos[...] += jax.lax.dot_general(
        onehot, pos_ref[...], (((1,), (0,)), ((), ())),
        preferred_element_type=jnp.float32)
    acc_cnt[...] += jax.lax.dot_general(
        onehot, jnp.ones((RC, 1), jnp.float32), (((1,), (0,)), ((), ())),
        preferred_element_type=jnp.float32)

    @pl.when(i == NC - 1)
    def _():
        center = acc_pos[...] / jnp.maximum(acc_cnt[...], 1.0)
        cadj_ref[...] = b1_ref[...] - jax.lax.dot_general(
            center, w1b_ref[...], (((1,), (0,)), ((), ())),
            preferred_element_type=jnp.float32)


def _main_body(bounds_ref, ids_ref, feat_ref, pos_ref, w1a_ref, w1b_ref,
               aggr_ref):
    i = pl.program_id(0)

    @pl.when(i == 0)
    def _():
        aggr_ref[...] = jnp.full((B, C_MID), _NEG_INF, jnp.float32)

    for t in range(NSUB):
        ids_t = ids_ref[0, t * SUB:(t + 1) * SUB]
        a_t = jax.lax.dot_general(
            feat_ref[t * SUB:(t + 1) * SUB, :].astype(jnp.bfloat16),
            w1a_ref[...], (((1,), (0,)), ((), ())),
            preferred_element_type=jnp.float32)
        a_t += jax.lax.dot_general(
            pos_ref[t * SUB:(t + 1) * SUB, :].astype(jnp.bfloat16),
            w1b_ref[...], (((1,), (0,)), ((), ())),
            preferred_element_type=jnp.float32)

        s_lo = bounds_ref[0, i * NSUB + t]
        s_hi = bounds_ref[1, i * NSUB + t]

        def seg_step(s, carry, ids_t=ids_t, a_t=a_t):
            red = jnp.max(jnp.where(ids_t == s, a_t, _NEG_INF), axis=0,
                          keepdims=True)
            aggr_ref[pl.ds(s, 1), :] = jnp.maximum(aggr_ref[pl.ds(s, 1), :],
                                                   red)
            return carry

        jax.lax.fori_loop(s_lo, s_hi + 1, seg_step, 0)


def _epi_body(aggr_ref, cadj_ref, w2_ref, b2_ref, eps_ref, z_ref, mu_ref, sig_ref):
    agg = jnp.maximum(aggr_ref[...] + cadj_ref[...], 0.0)
    out = jax.lax.dot_general(
        agg, w2_ref[...], (((1,), (0,)), ((), ())),
        preferred_element_type=jnp.float32) + b2_ref[...]
    mu = out[:, :C_MID]
    sr = out[:, C_MID:]
    sigma = jnp.maximum(sr, 0.0) + jnp.log1p(jnp.exp(-jnp.abs(sr)))
    mu_ref[...] = mu
    sig_ref[...] = sigma
    z_ref[...] = mu + sigma * eps_ref[...]


@functools.partial(jax.jit, static_argnames=("interpret",))
def _run(pos, feature, ids_col, ids_row, bounds, W1a, W1b, b1r, W2, b2r, eps,
         interpret=False):
    cadj = pl.pallas_call(
        _center_body,
        grid=(NC,),
        in_specs=[
            pl.BlockSpec((1, 1, RC), lambda i: (i, 0, 0)),
            pl.BlockSpec((RC, 3), lambda i: (i, 0)),
            pl.BlockSpec((3, C_MID), lambda i: (0, 0)),
            pl.BlockSpec((1, C_MID), lambda i: (0, 0)),
        ],
        out_specs=pl.BlockSpec((B, C_MID), lambda i: (0, 0)),
        out_shape=jax.ShapeDtypeStruct((B, C_MID), jnp.float32),
        scratch_shapes=[
            pltpu.VMEM((B, 3), jnp.float32),
            pltpu.VMEM((B, 1), jnp.float32),
        ],
        interpret=interpret,
    )(ids_row, pos, W1b, b1r)

    agg_raw = pl.pallas_call(
        _main_body,
        grid=(NB,),
        in_specs=[
            pl.BlockSpec(memory_space=pltpu.SMEM),
            pl.BlockSpec((1, R, 1), lambda i: (i, 0, 0)),
            pl.BlockSpec((R, C_IN), lambda i: (i, 0)),
            pl.BlockSpec((R, 3), lambda i: (i, 0)),
            pl.BlockSpec((C_IN, C_MID), lambda i: (0, 0)),
            pl.BlockSpec((3, C_MID), lambda i: (0, 0)),
        ],
        out_specs=pl.BlockSpec((B, C_MID), lambda i: (0, 0)),
        out_shape=jax.ShapeDtypeStruct((B, C_MID), jnp.float32),
        interpret=interpret,
    )(bounds, ids_col, feature, pos,
      W1a.astype(jnp.bfloat16), W1b.astype(jnp.bfloat16))

    z, mu, sigma = pl.pallas_call(
        _epi_body,
        in_specs=[
            pl.BlockSpec((B, C_MID), lambda: (0, 0)),
            pl.BlockSpec((B, C_MID), lambda: (0, 0)),
            pl.BlockSpec((C_MID, C_OUT), lambda: (0, 0)),
            pl.BlockSpec((1, C_OUT), lambda: (0, 0)),
            pl.BlockSpec((B, C_MID), lambda: (0, 0)),
        ],
        out_specs=[
            pl.BlockSpec((B, C_MID), lambda: (0, 0)),
            pl.BlockSpec((B, C_MID), lambda: (0, 0)),
            pl.BlockSpec((B, C_MID), lambda: (0, 0)),
        ],
        out_shape=[
            jax.ShapeDtypeStruct((B, C_MID), jnp.float32),
            jax.ShapeDtypeStruct((B, C_MID), jnp.float32),
            jax.ShapeDtypeStruct((B, C_MID), jnp.float32),
        ],
        interpret=interpret,
    )(agg_raw, cadj, W2, b2r, eps)
    return z, mu, sigma


def kernel(pos, feature, batch, W1, b1, W2, b2, *, interpret=False):
    ids = batch.astype(jnp.int32)
    ids_col = ids.reshape(NB, R, 1)
    ids_row = ids.reshape(NC, 1, RC)
    ids_sub = ids.reshape(NT, SUB)
    bounds = jnp.stack([ids_sub[:, 0], ids_sub[:, SUB - 1]])
    W1a = W1[:C_IN]
    W1b = W1[C_IN:]
    b1r = b1.reshape(1, C_MID)
    b2r = b2.reshape(1, C_OUT)
    eps = jax.random.normal(jax.random.key(1), (B, C_MID), dtype=jnp.float32)
    z, mu, sigma = _run(pos, feature, ids_col, ids_row, bounds, W1a, W1b, b1r,
                        W2, b2r, eps, interpret=interpret)
    pos_center_batch = jnp.arange(B, dtype=jnp.int64)
    return (z, mu, sigma, pos_center_batch)


# speedup vs baseline: 1.2141x; 1.1117x over previous
import functools

import jax
import jax.numpy as jnp
from jax.experimental import pallas as pl
from jax.experimental.pallas import tpu as pltpu

N = 100000
B = 64
C_IN = 256
C_MID = 256
C_OUT = 512

R = 2000
NB = N // R
SUB = 2000
NSUB = R // SUB
NT = N // SUB
RC = 5000
NC = N // RC

_NEG_INF = float("-inf")


def _center_body(ids_ref, pos_ref, w1b_ref, b1_ref, cadj_ref, acc_pos, acc_cnt):
    i = pl.program_id(0)

    @pl.when(i == 0)
    def _():
        acc_pos[...] = jnp.zeros_like(acc_pos)
        acc_cnt[...] = jnp.zeros_like(acc_cnt)

    ids_row = ids_ref[0]
    onehot = (jax.lax.broadcasted_iota(jnp.int32, (B, RC), 0) == ids_row
              ).astype(jnp.float32)
    acc_pos[...] += jax.lax.dot_general(
        onehot, pos_ref[...], (((1,), (0,)), ((), ())),
        preferred_element_type=jnp.float32)
    acc_cnt[...] += jax.lax.dot_general(
        onehot, jnp.ones((RC, 1), jnp.float32), (((1,), (0,)), ((), ())),
        preferred_element_type=jnp.float32)

    @pl.when(i == NC - 1)
    def _():
        center = acc_pos[...] / jnp.maximum(acc_cnt[...], 1.0)
        cadj_ref[...] = b1_ref[...] - jax.lax.dot_general(
            center, w1b_ref[...], (((1,), (0,)), ((), ())),
            preferred_element_type=jnp.float32)


def _main_body(ids_ref, feat_ref, pos_ref, w1a_ref, w1b_ref, aggr_ref):
    i = pl.program_id(0)

    @pl.when(i == 0)
    def _():
        aggr_ref[...] = jnp.full((B, C_MID), _NEG_INF, jnp.float32)

    ids = ids_ref[0]
    a = jax.lax.dot_general(
        feat_ref[...].astype(jnp.bfloat16), w1a_ref[...],
        (((1,), (0,)), ((), ())), preferred_element_type=jnp.float32)
    a += jax.lax.dot_general(
        pos_ref[...].astype(jnp.bfloat16), w1b_ref[...],
        (((1,), (0,)), ((), ())), preferred_element_type=jnp.float32)

    s_lo = ids_ref[0, 0, 0]
    s_hi = ids_ref[0, R - 1, 0]

    def seg_step(s, carry):
        red = jnp.max(jnp.where(ids == s, a, _NEG_INF), axis=0, keepdims=True)
        aggr_ref[pl.ds(s, 1), :] = jnp.maximum(aggr_ref[pl.ds(s, 1), :], red)
        return carry

    jax.lax.fori_loop(s_lo, s_hi + 1, seg_step, 0)


def _epi_body(aggr_ref, cadj_ref, w2_ref, b2_ref, eps_ref, z_ref, mu_ref, sig_ref):
    agg = jnp.maximum(aggr_ref[...] + cadj_ref[...], 0.0)
    out = jax.lax.dot_general(
        agg, w2_ref[...], (((1,), (0,)), ((), ())),
        preferred_element_type=jnp.float32) + b2_ref[...]
    mu = out[:, :C_MID]
    sr = out[:, C_MID:]
    sigma = jnp.maximum(sr, 0.0) + jnp.log1p(jnp.exp(-jnp.abs(sr)))
    mu_ref[...] = mu
    sig_ref[...] = sigma
    z_ref[...] = mu + sigma * eps_ref[...]


@functools.partial(jax.jit, static_argnames=("interpret",))
def _run(pos, feature, ids_col, ids_row, W1a, W1b, b1r, W2, b2r, eps,
         interpret=False):
    cadj = pl.pallas_call(
        _center_body,
        grid=(NC,),
        in_specs=[
            pl.BlockSpec((1, 1, RC), lambda i: (i, 0, 0)),
            pl.BlockSpec((RC, 3), lambda i: (i, 0)),
            pl.BlockSpec((3, C_MID), lambda i: (0, 0)),
            pl.BlockSpec((1, C_MID), lambda i: (0, 0)),
        ],
        out_specs=pl.BlockSpec((B, C_MID), lambda i: (0, 0)),
        out_shape=jax.ShapeDtypeStruct((B, C_MID), jnp.float32),
        scratch_shapes=[
            pltpu.VMEM((B, 3), jnp.float32),
            pltpu.VMEM((B, 1), jnp.float32),
        ],
        interpret=interpret,
    )(ids_row, pos, W1b, b1r)

    agg_raw = pl.pallas_call(
        _main_body,
        grid=(NB,),
        in_specs=[
            pl.BlockSpec((1, R, 1), lambda i: (i, 0, 0)),
            pl.BlockSpec((R, C_IN), lambda i: (i, 0)),
            pl.BlockSpec((R, 3), lambda i: (i, 0)),
            pl.BlockSpec((C_IN, C_MID), lambda i: (0, 0)),
            pl.BlockSpec((3, C_MID), lambda i: (0, 0)),
        ],
        out_specs=pl.BlockSpec((B, C_MID), lambda i: (0, 0)),
        out_shape=jax.ShapeDtypeStruct((B, C_MID), jnp.float32),
        interpret=interpret,
    )(ids_col, feature, pos,
      W1a.astype(jnp.bfloat16), W1b.astype(jnp.bfloat16))

    z, mu, sigma = pl.pallas_call(
        _epi_body,
        in_specs=[
            pl.BlockSpec((B, C_MID), lambda: (0, 0)),
            pl.BlockSpec((B, C_MID), lambda: (0, 0)),
            pl.BlockSpec((C_MID, C_OUT), lambda: (0, 0)),
            pl.BlockSpec((1, C_OUT), lambda: (0, 0)),
            pl.BlockSpec((B, C_MID), lambda: (0, 0)),
        ],
        out_specs=[
            pl.BlockSpec((B, C_MID), lambda: (0, 0)),
            pl.BlockSpec((B, C_MID), lambda: (0, 0)),
            pl.BlockSpec((B, C_MID), lambda: (0, 0)),
        ],
        out_shape=[
            jax.ShapeDtypeStruct((B, C_MID), jnp.float32),
            jax.ShapeDtypeStruct((B, C_MID), jnp.float32),
            jax.ShapeDtypeStruct((B, C_MID), jnp.float32),
        ],
        interpret=interpret,
    )(agg_raw, cadj, W2, b2r, eps)
    return z, mu, sigma


def kernel(pos, feature, batch, W1, b1, W2, b2, *, interpret=False):
    ids = batch.astype(jnp.int32)
    ids_col = ids.reshape(NB, R, 1)
    ids_row = ids.reshape(NC, 1, RC)
    W1a = W1[:C_IN]
    W1b = W1[C_IN:]
    b1r = b1.reshape(1, C_MID)
    b2r = b2.reshape(1, C_OUT)
    eps = jax.random.normal(jax.random.key(1), (B, C_MID), dtype=jnp.float32)
    z, mu, sigma = _run(pos, feature, ids_col, ids_row, W1a, W1b, b1r,
                        W2, b2r, eps, interpret=interpret)
    pos_center_batch = jnp.arange(B, dtype=jnp.int64)
    return (z, mu, sigma, pos_center_batch)
